# baseline (device time: 222829 ns/iter reference)
import jax
import jax.numpy as jnp
from jax import lax
from jax.experimental import pallas as pl
from jax.experimental.pallas import tpu as pltpu

N_DEV = 32
H_R = N_DEV // 2
H_L = N_DEV - 1 - H_R
SUBS = 4


def _hamiltonian_cycle():
    path_yz = []
    for z in range(4):
        ys = range(4) if z % 2 == 0 else range(3, -1, -1)
        path_yz.extend((y, z) for y in ys)
    coords = [(0, y, z) for (y, z) in path_yz]
    coords += [(1, y, z) for (y, z) in reversed(path_yz)]

    def logical(c):
        x, y, z = c
        return 8 * z + 2 * y + (x if y % 2 == 0 else 1 - x)

    cyc = [logical(c) for c in coords]
    assert sorted(cyc) == list(range(N_DEV))
    pos = [0] * N_DEV
    for p, l in enumerate(cyc):
        pos[l] = p
    return cyc, pos


_CYC, _POS = _hamiltonian_cycle()


def kernel(x):
    m_per, n = x.shape

    def body(cyc_ref, pos_ref, x_ref, out_ref, send_r, recv_r, send_l, recv_l):
        my_pos = lax.axis_index("i")
        p = pos_ref[my_pos]
        right = cyc_ref[lax.rem(p + 1, N_DEV)]
        left = cyc_ref[lax.rem(p + N_DEV - 1, N_DEV)]

        barrier_sem = pltpu.get_barrier_semaphore()
        for nbr in (left, right):
            pl.semaphore_signal(
                barrier_sem, inc=1,
                device_id=(nbr,), device_id_type=pl.DeviceIdType.MESH,
            )
        pl.semaphore_wait(barrier_sem, 2)

        out_ref[my_pos] = x_ref[...].astype(out_ref.dtype)

        m_sub = m_per // SUBS

        def rdma_hop(h, rightward, s):
            if rightward:
                o = cyc_ref[lax.rem(p - h + N_DEV, N_DEV)]
                tgt, ss, rs = right, send_r, recv_r
            else:
                o = cyc_ref[lax.rem(p + h, N_DEV)]
                tgt, ss, rs = left, send_l, recv_l
            sub = out_ref.at[o, pl.ds(s * m_sub, m_sub), :]
            return pltpu.make_async_remote_copy(
                src_ref=sub,
                dst_ref=sub,
                send_sem=ss.at[h, s],
                recv_sem=rs.at[h, s],
                device_id=(tgt,),
                device_id_type=pl.DeviceIdType.MESH,
            )

        for s in range(SUBS):
            rdma_hop(0, True, s).start()
            rdma_hop(0, False, s).start()
        for h in range(H_R):
            for s in range(SUBS):
                rdma_hop(h, True, s).wait_recv()
                if h + 1 < H_R:
                    rdma_hop(h + 1, True, s).start()
                if h < H_L:
                    rdma_hop(h, False, s).wait_recv()
                    if h + 1 < H_L:
                        rdma_hop(h + 1, False, s).start()

        for h in range(H_R):
            for s in range(SUBS):
                rdma_hop(h, True, s).wait_send()
        for h in range(H_L):
            for s in range(SUBS):
                rdma_hop(h, False, s).wait_send()

    out = pl.pallas_call(
        body,
        out_shape=jax.ShapeDtypeStruct((N_DEV, m_per, n), jnp.bfloat16),
        in_specs=[
            pl.BlockSpec(memory_space=pltpu.SMEM),
            pl.BlockSpec(memory_space=pltpu.SMEM),
            pl.BlockSpec(memory_space=pltpu.VMEM),
        ],
        out_specs=pl.BlockSpec(memory_space=pltpu.VMEM),
        scratch_shapes=[
            pltpu.SemaphoreType.DMA((H_R, SUBS)),
            pltpu.SemaphoreType.DMA((H_R, SUBS)),
            pltpu.SemaphoreType.DMA((H_L, SUBS)),
            pltpu.SemaphoreType.DMA((H_L, SUBS)),
        ],
        compiler_params=pltpu.CompilerParams(collective_id=0),
    )(
        jnp.asarray(_CYC, dtype=jnp.int32),
        jnp.asarray(_POS, dtype=jnp.int32),
        x,
    )
    return out.reshape(N_DEV * m_per, n)


# device time: 217507 ns/iter; 1.0245x vs baseline; 1.0245x over previous
import jax
import jax.numpy as jnp
from jax import lax
from jax.experimental import pallas as pl
from jax.experimental.pallas import tpu as pltpu

N_DEV = 32
N_HOP = N_DEV // 2
SUBS = 4


def _hamiltonian_cycle():
    path_yz = []
    for z in range(4):
        ys = range(4) if z % 2 == 0 else range(3, -1, -1)
        path_yz.extend((y, z) for y in ys)
    coords = [(0, y, z) for (y, z) in path_yz]
    coords += [(1, y, z) for (y, z) in reversed(path_yz)]

    def logical(c):
        x, y, z = c
        return 8 * z + 2 * y + (x if y % 2 == 0 else 1 - x)

    cyc = [logical(c) for c in coords]
    assert sorted(cyc) == list(range(N_DEV))
    pos = [0] * N_DEV
    for p, l in enumerate(cyc):
        pos[l] = p
    return cyc, pos


_CYC, _POS = _hamiltonian_cycle()


def _subs_at(h, rightward):
    if h < N_HOP - 1:
        return range(SUBS)
    return range(SUBS // 2) if rightward else range(SUBS // 2, SUBS)


def kernel(x):
    m_per, n = x.shape
    m_sub = m_per // SUBS

    def body(cyc_ref, pos_ref, x_ref, out_ref, send_r, recv_r, send_l, recv_l):
        my_pos = lax.axis_index("i")
        p = pos_ref[my_pos]
        right = cyc_ref[lax.rem(p + 1, N_DEV)]
        left = cyc_ref[lax.rem(p + N_DEV - 1, N_DEV)]

        barrier_sem = pltpu.get_barrier_semaphore()
        for nbr in (left, right):
            pl.semaphore_signal(
                barrier_sem, inc=1,
                device_id=(nbr,), device_id_type=pl.DeviceIdType.MESH,
            )
        pl.semaphore_wait(barrier_sem, 2)

        out_ref[pl.ds(my_pos * m_per, m_per), :] = x_ref[...].astype(
            out_ref.dtype
        )

        def rdma_hop(h, rightward, s):
            if rightward:
                o = cyc_ref[lax.rem(p - h + N_DEV, N_DEV)]
                tgt, ss, rs = right, send_r, recv_r
            else:
                o = cyc_ref[lax.rem(p + h, N_DEV)]
                tgt, ss, rs = left, send_l, recv_l
            sub = out_ref.at[pl.ds(o * m_per + s * m_sub, m_sub), :]
            return pltpu.make_async_remote_copy(
                src_ref=sub,
                dst_ref=sub,
                send_sem=ss.at[h, s],
                recv_sem=rs.at[h, s],
                device_id=(tgt,),
                device_id_type=pl.DeviceIdType.MESH,
            )

        for s in _subs_at(0, True):
            rdma_hop(0, True, s).start()
        for s in _subs_at(0, False):
            rdma_hop(0, False, s).start()
        for h in range(N_HOP):
            for s in range(SUBS):
                if s in _subs_at(h, True):
                    rdma_hop(h, True, s).wait_recv()
                if h + 1 < N_HOP and s in _subs_at(h + 1, True):
                    rdma_hop(h + 1, True, s).start()
                if s in _subs_at(h, False):
                    rdma_hop(h, False, s).wait_recv()
                if h + 1 < N_HOP and s in _subs_at(h + 1, False):
                    rdma_hop(h + 1, False, s).start()

        for h in range(N_HOP):
            for s in _subs_at(h, True):
                rdma_hop(h, True, s).wait_send()
            for s in _subs_at(h, False):
                rdma_hop(h, False, s).wait_send()

    return pl.pallas_call(
        body,
        out_shape=jax.ShapeDtypeStruct((N_DEV * m_per, n), jnp.bfloat16),
        in_specs=[
            pl.BlockSpec(memory_space=pltpu.SMEM),
            pl.BlockSpec(memory_space=pltpu.SMEM),
            pl.BlockSpec(memory_space=pltpu.VMEM),
        ],
        out_specs=pl.BlockSpec(memory_space=pltpu.VMEM),
        scratch_shapes=[
            pltpu.SemaphoreType.DMA((N_HOP, SUBS)),
            pltpu.SemaphoreType.DMA((N_HOP, SUBS)),
            pltpu.SemaphoreType.DMA((N_HOP, SUBS)),
            pltpu.SemaphoreType.DMA((N_HOP, SUBS)),
        ],
        compiler_params=pltpu.CompilerParams(collective_id=0),
    )(
        jnp.asarray(_CYC, dtype=jnp.int32),
        jnp.asarray(_POS, dtype=jnp.int32),
        x,
    )
